# speedup vs baseline: 8.5806x; 8.5806x over previous
import jax
import jax.numpy as jnp
from jax.experimental import pallas as pl

_BLOCK_ROWS = 256


def _fill_diag_block(fill_ref, x_ref, o_ref):
    i = pl.program_id(0)
    rows = jax.lax.broadcasted_iota(jnp.int32, x_ref.shape, 0) + i * _BLOCK_ROWS
    cols = jax.lax.broadcasted_iota(jnp.int32, x_ref.shape, 1)
    o_ref[...] = jnp.where(rows == cols, fill_ref[0, 0], x_ref[...])


def kernel(x, fill_value):
    n_rows, n_cols = x.shape
    fill = jnp.asarray(fill_value, x.dtype).reshape(1, 1)
    return pl.pallas_call(
        _fill_diag_block,
        grid=(n_rows // _BLOCK_ROWS,),
        in_specs=[
            pl.BlockSpec((1, 1), lambda i: (0, 0)),
            pl.BlockSpec((_BLOCK_ROWS, n_cols), lambda i: (i, 0)),
        ],
        out_specs=pl.BlockSpec((_BLOCK_ROWS, n_cols), lambda i: (i, 0)),
        out_shape=jax.ShapeDtypeStruct(x.shape, x.dtype),
    )(fill, x)
